# baseline (speedup 1.0000x reference)
import functools

import jax
import jax.numpy as jnp
from jax import lax
from jax.experimental import pallas as pl
from jax.experimental.pallas import tpu as pltpu
from jax.experimental.pallas import tpu_sc as plsc

N = 10000
E = 320000
NSC = 2
NTILE = 16
EB = 128
N_EDGE_CHUNKS = E // EB
DUMMY = 64
ROWS_PER_TILE = N // NTILE


P_CHUNKS = 2560


def _sc_aggregate(zs, src2p, dstp, f2, g):
    n_steps = P_CHUNKS // (NTILE * g)
    assert n_steps % 2 == 0
    mesh = plsc.VectorSubcoreMesh(core_axis_name="c", subcore_axis_name="s")

    @functools.partial(
        pl.kernel,
        out_type=jax.ShapeDtypeStruct((2 * N, f2), jnp.float32),
        mesh=mesh,
        compiler_params=pltpu.CompilerParams(use_tc_tiling_on_sc=False),
        scratch_types=[
            pltpu.VMEM_SHARED((N + DUMMY, f2), jnp.float32),
            pltpu.VMEM((2, g * EB), jnp.int32),
            pltpu.VMEM((2, g * EB), jnp.int32),
            pltpu.VMEM((2, g * EB, f2), jnp.float32),
            pltpu.SemaphoreType.DMA,
            pltpu.SemaphoreType.DMA,
            pltpu.SemaphoreType.DMA,
            pltpu.SemaphoreType.DMA,
        ],
    )
    def kfn(z_hbm, src2_hbm, dst_hbm, out_hbm, acc, sidx, didx, rows,
            g0, g1, s0, s1):
        c = lax.axis_index("c")
        t = lax.axis_index("s")
        gsems = (g0, g1)
        ssems = (s0, s1)

        def load_idx(step, rb):
            base = (step * NTILE + t) * g * EB
            pltpu.sync_copy(src2_hbm.at[c, pl.ds(base, g * EB)], sidx.at[rb])
            pltpu.sync_copy(dst_hbm.at[pl.ds(base, g * EB)], didx.at[rb])

        def fire_gather(rb):
            pltpu.async_copy(z_hbm.at[sidx.at[rb]], rows.at[rb], gsems[rb])

        def wait_gather(rb):
            pltpu.make_async_copy(z_hbm.at[sidx.at[rb]], rows.at[rb],
                                  gsems[rb]).wait()

        def fire_scatter(rb):
            pltpu.async_copy(rows.at[rb], acc.at[didx.at[rb]],
                             ssems[rb], add=True)

        def wait_scatter(rb):
            pltpu.make_async_copy(rows.at[rb], acc.at[didx.at[rb]],
                                  ssems[rb]).wait()

        load_idx(0, 0)
        fire_gather(0)

        r0 = t * ROWS_PER_TILE
        pltpu.sync_copy(z_hbm.at[pl.ds(c * N + r0, ROWS_PER_TILE)],
                        acc.at[pl.ds(r0, ROWS_PER_TILE)])
        plsc.subcore_barrier()

        def body(i, _):
            for par in (0, 1):
                s = 2 * i + par
                rb, ob = par, 1 - par
                @pl.when(s >= 1)
                def _():
                    wait_scatter(ob)

                @pl.when(s + 1 < n_steps)
                def _():
                    load_idx(s + 1, ob)
                    fire_gather(ob)

                wait_gather(rb)
                fire_scatter(rb)
            return 0

        lax.fori_loop(0, n_steps // 2, body, 0)
        wait_scatter((n_steps - 1) % 2)
        plsc.subcore_barrier()

        pltpu.sync_copy(acc.at[pl.ds(r0, ROWS_PER_TILE)],
                        out_hbm.at[pl.ds(c * N + r0, ROWS_PER_TILE)])

    return kfn(zs, src2p, dstp)


def _pad_edges(src, dst, p_chunks):
    pe = p_chunks * EB
    pad = pe - E
    srcp = jnp.concatenate([src, jnp.zeros((pad,), jnp.int32)])
    src2p = jnp.stack([srcp, srcp + N])
    dstp = jnp.concatenate(
        [dst, N + (jnp.arange(pad, dtype=jnp.int32) % DUMMY)]
    )
    return src2p, dstp


TC_RB = 400


def _tc_layer1(s1, W1, b1, W2):
    f2 = W2.shape[1] // 2

    def body(s_ref, w1_ref, b1_ref, w2_ref, o_ref):
        h = jnp.concatenate([s_ref[0], s_ref[1]], axis=1)
        h1 = jax.nn.relu(jnp.dot(h, w1_ref[...],
                                 preferred_element_type=jnp.float32) + b1_ref[...])
        z = jnp.dot(h1, w2_ref[...], preferred_element_type=jnp.float32)
        o_ref[0] = z[:, :f2]
        o_ref[1] = z[:, f2:]

    return pl.pallas_call(
        body,
        grid=(N // TC_RB,),
        in_specs=[
            pl.BlockSpec((2, TC_RB, s1.shape[2]), lambda r: (0, r, 0)),
            pl.BlockSpec(W1.shape, lambda r: (0, 0)),
            pl.BlockSpec(b1.shape, lambda r: (0, 0)),
            pl.BlockSpec(W2.shape, lambda r: (0, 0)),
        ],
        out_specs=pl.BlockSpec((2, TC_RB, f2), lambda r: (0, r, 0)),
        out_shape=jax.ShapeDtypeStruct((2, N, f2), jnp.float32),
    )(s1, W1, b1, W2)


def _tc_mid(s, b, W):
    f2 = W.shape[1] // 2

    def body(s_ref, b_ref, w_ref, o_ref):
        h = jax.nn.relu(jnp.concatenate([s_ref[0], s_ref[1]], axis=1)
                        + b_ref[...])
        z = jnp.dot(h, w_ref[...], preferred_element_type=jnp.float32)
        o_ref[0] = z[:, :f2]
        o_ref[1] = z[:, f2:]

    return pl.pallas_call(
        body,
        grid=(N // TC_RB,),
        in_specs=[
            pl.BlockSpec((2, TC_RB, s.shape[2]), lambda r: (0, r, 0)),
            pl.BlockSpec(b.shape, lambda r: (0, 0)),
            pl.BlockSpec(W.shape, lambda r: (0, 0)),
        ],
        out_specs=pl.BlockSpec((2, TC_RB, f2), lambda r: (0, r, 0)),
        out_shape=jax.ShapeDtypeStruct((2, N, f2), jnp.float32),
    )(s, b, W)


def _tc_logsoftmax(s5, b5, n_cls):
    def body(s_ref, b_ref, o_ref):
        y = jnp.concatenate([s_ref[0], s_ref[1]], axis=1)[:, :n_cls] + b_ref[...]
        m = jnp.max(y, axis=1, keepdims=True)
        e = jnp.exp(y - m)
        o_ref[...] = y - m - jnp.log(jnp.sum(e, axis=1, keepdims=True))

    return pl.pallas_call(
        body,
        grid=(N // TC_RB,),
        in_specs=[
            pl.BlockSpec((2, TC_RB, s5.shape[2]), lambda r: (0, r, 0)),
            pl.BlockSpec(b5.shape, lambda r: (0, 0)),
        ],
        out_specs=pl.BlockSpec((TC_RB, n_cls), lambda r: (r, 0)),
        out_shape=jax.ShapeDtypeStruct((N, n_cls), jnp.float32),
    )(s5, b5)


def kernel(x, edge_index, W1, b1, W2, b2, W3, b3, W4, b4, W5, b5):
    n_cls = W5.shape[1]
    src = edge_index[0]
    dst = edge_index[1]

    src2p, dstp = _pad_edges(src, dst, P_CHUNKS)

    W5p = jnp.pad(W5, ((0, 0), (0, 64 - n_cls)))

    b1r = b1.reshape(1, -1)
    b2r = b2.reshape(1, -1)
    b3r = b3.reshape(1, -1)
    b4r = b4.reshape(1, -1)
    b5r = b5.reshape(1, -1)

    x2 = jnp.concatenate([x[:, :64], x[:, 64:]], axis=0)
    s1 = _sc_aggregate(x2, src2p, dstp, 64, 4)
    z2 = _tc_layer1(s1.reshape(2, N, 64), W1, b1r, W2)

    s2 = _sc_aggregate(z2.reshape(2 * N, 128), src2p, dstp, 128, 1)
    z3 = _tc_mid(s2.reshape(2, N, 128), b2r, W3)

    s3 = _sc_aggregate(z3.reshape(2 * N, 128), src2p, dstp, 128, 1)
    z4 = _tc_mid(s3.reshape(2, N, 128), b3r, W4)

    s4 = _sc_aggregate(z4.reshape(2 * N, 128), src2p, dstp, 128, 1)
    z5 = _tc_mid(s4.reshape(2, N, 128), b4r, W5p)

    s5 = _sc_aggregate(z5.reshape(2 * N, 32), src2p, dstp, 32, 8)
    return _tc_logsoftmax(s5.reshape(2, N, 32), b5r, n_cls)


# speedup vs baseline: 1.1008x; 1.1008x over previous
import functools

import jax
import jax.numpy as jnp
from jax import lax
from jax.experimental import pallas as pl
from jax.experimental.pallas import tpu as pltpu
from jax.experimental.pallas import tpu_sc as plsc

N = 10000
E = 320000
NSC = 2
NTILE = 16
EB = 128
N_EDGE_CHUNKS = E // EB
DUMMY = 64
ROWS_PER_TILE = N // NTILE


P_CHUNKS = 2560


def _sc_aggregate(zs, src2p, dstp, f2, ch_g):
    ch, g = ch_g
    assert ch % g == 0
    ncmd = ch // g
    assert ncmd % 2 == 0
    n_steps = P_CHUNKS // (NTILE * ch)
    assert n_steps % 2 == 0 and P_CHUNKS % (NTILE * ch) == 0
    mesh = plsc.VectorSubcoreMesh(core_axis_name="c", subcore_axis_name="s")

    @functools.partial(
        pl.kernel,
        out_type=jax.ShapeDtypeStruct((2 * N, f2), jnp.float32),
        mesh=mesh,
        compiler_params=pltpu.CompilerParams(use_tc_tiling_on_sc=False),
        scratch_types=[
            pltpu.VMEM_SHARED((N + DUMMY, f2), jnp.float32),
            pltpu.VMEM((2, ncmd, g * EB), jnp.int32),
            pltpu.VMEM((2, ncmd, g * EB), jnp.int32),
            pltpu.VMEM((2, g * EB, f2), jnp.float32),
            pltpu.SemaphoreType.DMA,
            pltpu.SemaphoreType.DMA,
            pltpu.SemaphoreType.DMA,
            pltpu.SemaphoreType.DMA,
        ],
    )
    def kfn(z_hbm, src2_hbm, dst_hbm, out_hbm, acc, sidx, didx, rows,
            g0, g1, s0, s1):
        c = lax.axis_index("c")
        t = lax.axis_index("s")
        gsems = (g0, g1)
        ssems = (s0, s1)

        def load_idx(step, set_i):
            base = (step * NTILE + t) * ncmd
            pltpu.sync_copy(src2_hbm.at[c, pl.ds(base, ncmd)], sidx.at[set_i])
            pltpu.sync_copy(dst_hbm.at[pl.ds(base, ncmd)], didx.at[set_i])

        def idx_slice(buf, set_i, m):
            return buf.at[set_i, m]

        def fire_gather(set_i, m, rb):
            pltpu.async_copy(z_hbm.at[idx_slice(sidx, set_i, m)],
                             rows.at[rb], gsems[rb])

        def wait_gather(set_i, m, rb):
            pltpu.make_async_copy(z_hbm.at[idx_slice(sidx, set_i, m)],
                                  rows.at[rb], gsems[rb]).wait()

        def fire_scatter(set_i, m, rb):
            pltpu.async_copy(rows.at[rb], acc.at[idx_slice(didx, set_i, m)],
                             ssems[rb], add=True)

        def wait_scatter(set_i, m, rb):
            pltpu.make_async_copy(rows.at[rb],
                                  acc.at[idx_slice(didx, set_i, m)],
                                  ssems[rb]).wait()

        load_idx(0, 0)
        fire_gather(0, 0, 0)

        r0 = t * ROWS_PER_TILE
        pltpu.sync_copy(z_hbm.at[pl.ds(c * N + r0, ROWS_PER_TILE)],
                        acc.at[pl.ds(r0, ROWS_PER_TILE)])
        plsc.subcore_barrier()

        def body(i, _):
            for set_i in (0, 1):
                s = 2 * i + set_i
                for m in range(ncmd):
                    rb = m % 2
                    if m == 0:
                        @pl.when(s >= 1)
                        def _():
                            wait_scatter(1 - set_i, ncmd - 1, 1 - rb)

                        @pl.when(s + 1 < n_steps)
                        def _():
                            load_idx(s + 1, 1 - set_i)
                    else:
                        wait_scatter(set_i, m - 1, 1 - rb)
                    if m < ncmd - 1:
                        fire_gather(set_i, m + 1, 1 - rb)
                    else:
                        @pl.when(s + 1 < n_steps)
                        def _():
                            fire_gather(1 - set_i, 0, 1 - rb)
                    wait_gather(set_i, m, rb)
                    fire_scatter(set_i, m, rb)
            return 0

        lax.fori_loop(0, n_steps // 2, body, 0)
        wait_scatter((n_steps - 1) % 2, ncmd - 1, (ncmd - 1) % 2)
        plsc.subcore_barrier()

        pltpu.sync_copy(acc.at[pl.ds(r0, ROWS_PER_TILE)],
                        out_hbm.at[pl.ds(c * N + r0, ROWS_PER_TILE)])

    return kfn(zs, src2p, dstp)


def _pad_edges(src, dst, p_chunks):
    pe = p_chunks * EB
    pad = pe - E
    srcp = jnp.concatenate([src, jnp.zeros((pad,), jnp.int32)])
    src2p = jnp.stack([srcp, srcp + N])
    dstp = jnp.concatenate(
        [dst, N + (jnp.arange(pad, dtype=jnp.int32) % DUMMY)]
    )
    return src2p, dstp


TC_RB = 400


def _tc_layer1(s1, W1, b1, W2):
    f2 = W2.shape[1] // 2

    def body(s_ref, w1_ref, b1_ref, w2_ref, o_ref):
        h = jnp.concatenate([s_ref[0], s_ref[1]], axis=1)
        h1 = jax.nn.relu(jnp.dot(h, w1_ref[...],
                                 preferred_element_type=jnp.float32) + b1_ref[...])
        z = jnp.dot(h1, w2_ref[...], preferred_element_type=jnp.float32)
        o_ref[0] = z[:, :f2]
        o_ref[1] = z[:, f2:]

    return pl.pallas_call(
        body,
        grid=(N // TC_RB,),
        in_specs=[
            pl.BlockSpec((2, TC_RB, s1.shape[2]), lambda r: (0, r, 0)),
            pl.BlockSpec(W1.shape, lambda r: (0, 0)),
            pl.BlockSpec(b1.shape, lambda r: (0, 0)),
            pl.BlockSpec(W2.shape, lambda r: (0, 0)),
        ],
        out_specs=pl.BlockSpec((2, TC_RB, f2), lambda r: (0, r, 0)),
        out_shape=jax.ShapeDtypeStruct((2, N, f2), jnp.float32),
    )(s1, W1, b1, W2)


def _tc_mid(s, b, W):
    f2 = W.shape[1] // 2

    def body(s_ref, b_ref, w_ref, o_ref):
        h = jax.nn.relu(jnp.concatenate([s_ref[0], s_ref[1]], axis=1)
                        + b_ref[...])
        z = jnp.dot(h, w_ref[...], preferred_element_type=jnp.float32)
        o_ref[0] = z[:, :f2]
        o_ref[1] = z[:, f2:]

    return pl.pallas_call(
        body,
        grid=(N // TC_RB,),
        in_specs=[
            pl.BlockSpec((2, TC_RB, s.shape[2]), lambda r: (0, r, 0)),
            pl.BlockSpec(b.shape, lambda r: (0, 0)),
            pl.BlockSpec(W.shape, lambda r: (0, 0)),
        ],
        out_specs=pl.BlockSpec((2, TC_RB, f2), lambda r: (0, r, 0)),
        out_shape=jax.ShapeDtypeStruct((2, N, f2), jnp.float32),
    )(s, b, W)


def _tc_logsoftmax(s5, b5, n_cls):
    def body(s_ref, b_ref, o_ref):
        y = jnp.concatenate([s_ref[0], s_ref[1]], axis=1)[:, :n_cls] + b_ref[...]
        m = jnp.max(y, axis=1, keepdims=True)
        e = jnp.exp(y - m)
        o_ref[...] = y - m - jnp.log(jnp.sum(e, axis=1, keepdims=True))

    return pl.pallas_call(
        body,
        grid=(N // TC_RB,),
        in_specs=[
            pl.BlockSpec((2, TC_RB, s5.shape[2]), lambda r: (0, r, 0)),
            pl.BlockSpec(b5.shape, lambda r: (0, 0)),
        ],
        out_specs=pl.BlockSpec((TC_RB, n_cls), lambda r: (r, 0)),
        out_shape=jax.ShapeDtypeStruct((N, n_cls), jnp.float32),
    )(s5, b5)


def kernel(x, edge_index, W1, b1, W2, b2, W3, b3, W4, b4, W5, b5):
    n_cls = W5.shape[1]
    src = edge_index[0]
    dst = edge_index[1]

    src2p, dstp = _pad_edges(src, dst, P_CHUNKS)

    W5p = jnp.pad(W5, ((0, 0), (0, 64 - n_cls)))

    b1r = b1.reshape(1, -1)
    b2r = b2.reshape(1, -1)
    b3r = b3.reshape(1, -1)
    b4r = b4.reshape(1, -1)
    b5r = b5.reshape(1, -1)

    x2 = jnp.concatenate([x[:, :64], x[:, 64:]], axis=0)
    def views(g):
        return (src2p.reshape(2, -1, g * EB), dstp.reshape(-1, g * EB))

    s4v, d4v = views(4)
    s1v, d1v = views(1)
    s8v, d8v = views(8)

    s1 = _sc_aggregate(x2, s4v, d4v, 64, (8, 4))
    z2 = _tc_layer1(s1.reshape(2, N, 64), W1, b1r, W2)

    s2 = _sc_aggregate(z2.reshape(2 * N, 128), s1v, d1v, 128, (8, 1))
    z3 = _tc_mid(s2.reshape(2, N, 128), b2r, W3)

    s3 = _sc_aggregate(z3.reshape(2 * N, 128), s1v, d1v, 128, (8, 1))
    z4 = _tc_mid(s3.reshape(2, N, 128), b3r, W4)

    s4 = _sc_aggregate(z4.reshape(2 * N, 128), s1v, d1v, 128, (8, 1))
    z5 = _tc_mid(s4.reshape(2, N, 128), b4r, W5p)

    s5 = _sc_aggregate(z5.reshape(2 * N, 32), s8v, d8v, 32, (16, 8))
    return _tc_logsoftmax(s5.reshape(2, N, 32), b5r, n_cls)


# speedup vs baseline: 1.1015x; 1.0006x over previous
import functools

import jax
import jax.numpy as jnp
from jax import lax
from jax.experimental import pallas as pl
from jax.experimental.pallas import tpu as pltpu
from jax.experimental.pallas import tpu_sc as plsc

N = 10000
E = 320000
NSC = 2
NTILE = 16
EB = 128
N_EDGE_CHUNKS = E // EB
DUMMY = 64
ROWS_PER_TILE = N // NTILE
R_MAIN = 632
R_LAST = N - R_MAIN * (NTILE - 1)


P_CHUNKS = 2560


def _sc_aggregate(zs, src2p, dstp, f2, ch_g):
    ch, g = ch_g
    assert ch % g == 0
    ncmd = ch // g
    assert ncmd % 2 == 0
    n_steps = P_CHUNKS // (NTILE * ch)
    assert n_steps % 2 == 0 and P_CHUNKS % (NTILE * ch) == 0
    mesh = plsc.VectorSubcoreMesh(core_axis_name="c", subcore_axis_name="s")

    @functools.partial(
        pl.kernel,
        out_type=jax.ShapeDtypeStruct((2 * N, f2), jnp.float32),
        mesh=mesh,
        compiler_params=pltpu.CompilerParams(use_tc_tiling_on_sc=(f2 == 128)),
        scratch_types=[
            pltpu.VMEM_SHARED((N + DUMMY, f2), jnp.float32),
            pltpu.VMEM((2, ncmd, g * EB), jnp.int32),
            pltpu.VMEM((2, ncmd, g * EB), jnp.int32),
            pltpu.VMEM((2, g * EB, f2), jnp.float32),
            pltpu.SemaphoreType.DMA,
            pltpu.SemaphoreType.DMA,
            pltpu.SemaphoreType.DMA,
            pltpu.SemaphoreType.DMA,
        ],
    )
    def kfn(z_hbm, src2_hbm, dst_hbm, out_hbm, acc, sidx, didx, rows,
            g0, g1, s0, s1):
        c = lax.axis_index("c")
        t = lax.axis_index("s")
        gsems = (g0, g1)
        ssems = (s0, s1)

        def load_idx(step, set_i):
            base = pl.multiple_of((step * NTILE + t) * ncmd, ncmd)
            pltpu.sync_copy(src2_hbm.at[c, pl.ds(base, ncmd)], sidx.at[set_i])
            pltpu.sync_copy(dst_hbm.at[pl.ds(base, ncmd)], didx.at[set_i])

        def idx_slice(buf, set_i, m):
            return buf.at[set_i, m]

        def fire_gather(set_i, m, rb):
            pltpu.async_copy(z_hbm.at[idx_slice(sidx, set_i, m)],
                             rows.at[rb], gsems[rb])

        def wait_gather(set_i, m, rb):
            pltpu.make_async_copy(z_hbm.at[idx_slice(sidx, set_i, m)],
                                  rows.at[rb], gsems[rb]).wait()

        def fire_scatter(set_i, m, rb):
            pltpu.async_copy(rows.at[rb], acc.at[idx_slice(didx, set_i, m)],
                             ssems[rb], add=True)

        def wait_scatter(set_i, m, rb):
            pltpu.make_async_copy(rows.at[rb],
                                  acc.at[idx_slice(didx, set_i, m)],
                                  ssems[rb]).wait()

        load_idx(0, 0)
        fire_gather(0, 0, 0)

        r0 = pl.multiple_of(t * R_MAIN, 8)

        @pl.when(t < NTILE - 1)
        def _():
            pltpu.sync_copy(z_hbm.at[pl.ds(c * N + r0, R_MAIN)],
                            acc.at[pl.ds(r0, R_MAIN)])

        @pl.when(t == NTILE - 1)
        def _():
            pltpu.sync_copy(z_hbm.at[pl.ds(c * N + r0, R_LAST)],
                            acc.at[pl.ds(r0, R_LAST)])

        plsc.subcore_barrier()

        def body(i, _):
            for set_i in (0, 1):
                s = 2 * i + set_i
                for m in range(ncmd):
                    rb = m % 2
                    if m == 0:
                        @pl.when(s >= 1)
                        def _():
                            wait_scatter(1 - set_i, ncmd - 1, 1 - rb)

                        @pl.when(s + 1 < n_steps)
                        def _():
                            load_idx(s + 1, 1 - set_i)
                    else:
                        wait_scatter(set_i, m - 1, 1 - rb)
                    if m < ncmd - 1:
                        fire_gather(set_i, m + 1, 1 - rb)
                    else:
                        @pl.when(s + 1 < n_steps)
                        def _():
                            fire_gather(1 - set_i, 0, 1 - rb)
                    wait_gather(set_i, m, rb)
                    fire_scatter(set_i, m, rb)
            return 0

        lax.fori_loop(0, n_steps // 2, body, 0)
        wait_scatter((n_steps - 1) % 2, ncmd - 1, (ncmd - 1) % 2)
        plsc.subcore_barrier()

        @pl.when(t < NTILE - 1)
        def _():
            pltpu.sync_copy(acc.at[pl.ds(r0, R_MAIN)],
                            out_hbm.at[pl.ds(c * N + r0, R_MAIN)])

        @pl.when(t == NTILE - 1)
        def _():
            pltpu.sync_copy(acc.at[pl.ds(r0, R_LAST)],
                            out_hbm.at[pl.ds(c * N + r0, R_LAST)])

    return kfn(zs, src2p, dstp)


def _pad_edges(src, dst, p_chunks):
    pe = p_chunks * EB
    pad = pe - E
    srcp = jnp.concatenate([src, jnp.zeros((pad,), jnp.int32)])
    src2p = jnp.stack([srcp, srcp + N])
    dstp = jnp.concatenate(
        [dst, N + (jnp.arange(pad, dtype=jnp.int32) % DUMMY)]
    )
    return src2p, dstp


TC_RB = 400


def _tc_layer1(s1, W1, b1, W2):
    f2 = W2.shape[1] // 2

    def body(s_ref, w1_ref, b1_ref, w2_ref, o_ref):
        h = jnp.concatenate([s_ref[0], s_ref[1]], axis=1)
        h1 = jax.nn.relu(jnp.dot(h, w1_ref[...],
                                 preferred_element_type=jnp.float32) + b1_ref[...])
        z = jnp.dot(h1, w2_ref[...], preferred_element_type=jnp.float32)
        o_ref[0] = z[:, :f2]
        o_ref[1] = z[:, f2:]

    return pl.pallas_call(
        body,
        grid=(N // TC_RB,),
        in_specs=[
            pl.BlockSpec((2, TC_RB, s1.shape[2]), lambda r: (0, r, 0)),
            pl.BlockSpec(W1.shape, lambda r: (0, 0)),
            pl.BlockSpec(b1.shape, lambda r: (0, 0)),
            pl.BlockSpec(W2.shape, lambda r: (0, 0)),
        ],
        out_specs=pl.BlockSpec((2, TC_RB, f2), lambda r: (0, r, 0)),
        out_shape=jax.ShapeDtypeStruct((2, N, f2), jnp.float32),
    )(s1, W1, b1, W2)


def _tc_mid(s, b, W):
    f2 = W.shape[1] // 2

    def body(s_ref, b_ref, w_ref, o_ref):
        h = jax.nn.relu(jnp.concatenate([s_ref[0], s_ref[1]], axis=1)
                        + b_ref[...])
        z = jnp.dot(h, w_ref[...], preferred_element_type=jnp.float32)
        o_ref[0] = z[:, :f2]
        o_ref[1] = z[:, f2:]

    return pl.pallas_call(
        body,
        grid=(N // TC_RB,),
        in_specs=[
            pl.BlockSpec((2, TC_RB, s.shape[2]), lambda r: (0, r, 0)),
            pl.BlockSpec(b.shape, lambda r: (0, 0)),
            pl.BlockSpec(W.shape, lambda r: (0, 0)),
        ],
        out_specs=pl.BlockSpec((2, TC_RB, f2), lambda r: (0, r, 0)),
        out_shape=jax.ShapeDtypeStruct((2, N, f2), jnp.float32),
    )(s, b, W)


def _tc_logsoftmax(s5, b5, n_cls):
    def body(s_ref, b_ref, o_ref):
        y = jnp.concatenate([s_ref[0], s_ref[1]], axis=1)[:, :n_cls] + b_ref[...]
        m = jnp.max(y, axis=1, keepdims=True)
        e = jnp.exp(y - m)
        o_ref[...] = y - m - jnp.log(jnp.sum(e, axis=1, keepdims=True))

    return pl.pallas_call(
        body,
        grid=(N // TC_RB,),
        in_specs=[
            pl.BlockSpec((2, TC_RB, s5.shape[2]), lambda r: (0, r, 0)),
            pl.BlockSpec(b5.shape, lambda r: (0, 0)),
        ],
        out_specs=pl.BlockSpec((TC_RB, n_cls), lambda r: (r, 0)),
        out_shape=jax.ShapeDtypeStruct((N, n_cls), jnp.float32),
    )(s5, b5)


def kernel(x, edge_index, W1, b1, W2, b2, W3, b3, W4, b4, W5, b5):
    n_cls = W5.shape[1]
    src = edge_index[0]
    dst = edge_index[1]

    src2p, dstp = _pad_edges(src, dst, P_CHUNKS)

    W5p = jnp.pad(W5, ((0, 0), (0, 64 - n_cls)))

    b1r = b1.reshape(1, -1)
    b2r = b2.reshape(1, -1)
    b3r = b3.reshape(1, -1)
    b4r = b4.reshape(1, -1)
    b5r = b5.reshape(1, -1)

    x2 = jnp.concatenate([x[:, :64], x[:, 64:]], axis=0)
    def views(g):
        return (src2p.reshape(2, -1, g * EB), dstp.reshape(-1, g * EB))

    s4v, d4v = views(4)
    s1v, d1v = views(1)
    s8v, d8v = views(8)

    s1 = _sc_aggregate(x2, s4v, d4v, 64, (8, 4))
    z2 = _tc_layer1(s1.reshape(2, N, 64), W1, b1r, W2)

    s2 = _sc_aggregate(z2.reshape(2 * N, 128), s1v, d1v, 128, (8, 1))
    z3 = _tc_mid(s2.reshape(2, N, 128), b2r, W3)

    s3 = _sc_aggregate(z3.reshape(2 * N, 128), s1v, d1v, 128, (8, 1))
    z4 = _tc_mid(s3.reshape(2, N, 128), b3r, W4)

    s4 = _sc_aggregate(z4.reshape(2 * N, 128), s1v, d1v, 128, (8, 1))
    z5 = _tc_mid(s4.reshape(2, N, 128), b4r, W5p)

    s5 = _sc_aggregate(z5.reshape(2 * N, 32), s8v, d8v, 32, (16, 8))
    return _tc_logsoftmax(s5.reshape(2, N, 32), b5r, n_cls)
